# baseline (speedup 1.0000x reference)
import functools

import jax
import jax.numpy as jnp
from jax import lax
from jax.experimental import pallas as pl
from jax.experimental.pallas import tpu as pltpu
from jax.experimental.pallas import tpu_sc as plsc

H = 256
AD = 128
NCELL = 10000
NCLUSTER = 2000
E = 160000
NW = 32
PW = 5120
EP = NW * PW
NCHUNK = PW // 16
HBR = 79
HBINS = HBR * 128
CW = 125
NJ = 80
WIN = 8
ACCR = 10112
STRIPE = ACCR // 16
RB = 1000


def _dg(a, b, ca, cb):
    return lax.dot_general(a, b, dimension_numbers=(((ca,), (cb,)), ((), ())),
                           preferred_element_type=jnp.float32)


def _cluster_tc(ch_ref, wd_ref, wl_ref, wg_ref, wc_ref, wm_ref, wupd_ref,
                wcfc_ref, wclfc_ref, wattn_ref,
                pm_ref, u_ref, cp_ref, cr_ref, fcv_ref):
    ch = ch_ref[...]
    for t, w_ref in enumerate((wd_ref, wl_ref, wg_ref, wc_ref)):
        vs = wupd_ref[:, (t + 1) * H:(t + 2) * H]
        cp_ref[t] = _dg(w_ref[:, :H], vs, 0, 1)
        cr_ref[t] = _dg(w_ref[:, H:], vs, 0, 1)
    v5 = wupd_ref[:, 5 * H:6 * H]
    cm = _dg(wm_ref[:, :H], v5, 0, 1)
    cr_ref[4] = _dg(wm_ref[:, H:], v5, 0, 1)
    pm = _dg(ch, cm, 1, 0)
    pm_ref[0] = pm[:, :AD]
    pm_ref[1] = pm[:, AD:]
    z = _dg(ch, wclfc_ref[...], 1, 1)
    u_ref[...] = _dg(z, wattn_ref[:, :AD], 1, 1)
    fcv_ref[...] = _dg(wcfc_ref[...], wattn_ref[:, AD:], 0, 1)


def _run_cluster_tc(cluster_h, wd, wl, wg, wc, w_may, w_upd,
                    w_cell_fc, w_cluster_fc, w_attn):
    return pl.pallas_call(
        _cluster_tc,
        out_shape=[
            jax.ShapeDtypeStruct((2, NCLUSTER, AD), jnp.float32),
            jax.ShapeDtypeStruct((NCLUSTER, 1), jnp.float32),
            jax.ShapeDtypeStruct((4, H, H), jnp.float32),
            jax.ShapeDtypeStruct((5, H, H), jnp.float32),
            jax.ShapeDtypeStruct((H, 1), jnp.float32),
        ],
    )(cluster_h, wd, wl, wg, wc, w_may, w_upd,
      w_cell_fc, w_cluster_fc, w_attn)


def _vcell_tc(x_ref, fcv_ref, v_ref):
    v_ref[...] = _dg(x_ref[...], fcv_ref[...], 1, 0)


def _run_vcell_tc(cell_h, fcv):
    return pl.pallas_call(
        _vcell_tc,
        out_shape=jax.ShapeDtypeStruct((NCELL, 1), jnp.float32),
    )(cell_h, fcv)


def _cell_tc(x_ref, cp_ref, cr_ref, wupd_ref, p_ref, r_ref):
    x = x_ref[...]
    r_ref[0] = _dg(x, wupd_ref[:, 0:H], 1, 1)
    for t in range(4):
        p = _dg(x, cp_ref[t], 1, 0)
        p_ref[t, 0] = p[:, :AD]
        p_ref[t, 1] = p[:, AD:]
    for t in range(5):
        r_ref[t + 1] = _dg(x, cr_ref[t], 1, 0)


def _run_cell_tc(cell_h, cp, cr, w_upd):
    nb = NCELL // RB
    full = lambda s: pl.BlockSpec(s, lambda i: tuple(0 for _ in s))
    return pl.pallas_call(
        _cell_tc,
        grid=(nb,),
        in_specs=[
            pl.BlockSpec((RB, H), lambda i: (i, 0)),
            full((4, H, H)), full((5, H, H)), full((H, 6 * H)),
        ],
        out_specs=[
            pl.BlockSpec((4, 2, RB, AD), lambda i: (0, 0, i, 0)),
            pl.BlockSpec((6, RB, H), lambda i: (0, i, 0)),
        ],
        out_shape=[
            jax.ShapeDtypeStruct((4, 2, NCELL, AD), jnp.float32),
            jax.ShapeDtypeStruct((6, NCELL, H), jnp.float32),
        ],
    )(cell_h, cp, cr, w_upd)


def _sc_pass1(dsts, srcs, u, vpad):
    mesh = plsc.VectorSubcoreMesh(core_axis_name="c", subcore_axis_name="s")

    @functools.partial(
        pl.kernel, mesh=mesh,
        compiler_params=pltpu.CompilerParams(use_tc_tiling_on_sc=False,
                                             needs_layout_passes=False),
        out_type=[
            jax.ShapeDtypeStruct((NW * 5 * HBINS,), jnp.float32),
            jax.ShapeDtypeStruct((NW, PW // 128, 128), jnp.float32),
        ],
        scratch_types=[
            pltpu.VMEM((PW // 128, 128), jnp.int32),
            pltpu.VMEM((PW // 128, 128), jnp.int32),
            pltpu.VMEM((PW // 128, 128), jnp.float32),
            pltpu.VMEM((NCLUSTER,), jnp.float32),
            pltpu.VMEM((HBINS,), jnp.float32),
            pltpu.VMEM((HBINS,), jnp.float32),
            pltpu.VMEM((HBINS,), jnp.float32),
            pltpu.VMEM((HBINS,), jnp.float32),
            pltpu.VMEM((HBINS,), jnp.float32),
            pltpu.VMEM((HBINS,), jnp.float32),
        ],
    )
    def k(dsts_hbm, srcs_hbm, u_hbm, v_hbm, cnt_hbm, ev_hbm,
          ibuf, sbuf, ebuf, u_tab, v_tab, h0, h1, h2, h3, h4):
        hists = (h0, h1, h2, h3, h4)
        wid = lax.axis_index("s") * 2 + lax.axis_index("c")

        zeros16 = jnp.zeros((16,), jnp.float32)

        def zero_body(i, _):
            for hr in hists:
                hr[pl.ds(i * 16, 16)] = zeros16
            return 0
        lax.fori_loop(0, HBINS // 16, zero_body, 0)

        pltpu.sync_copy(u_hbm, u_tab)
        pltpu.sync_copy(v_hbm, v_tab)

        ones16 = jnp.ones((16,), jnp.float32)
        for t in range(4):
            pltpu.sync_copy(dsts_hbm.at[t, wid], ibuf)

            def cbody(i, _, _t=t):
                r = i // 8
                kk = i % 8
                d = ibuf[r, pl.ds(kk * 16, 16)]
                plsc.addupdate_scatter(hists[_t], [d], ones16)
                return 0
            lax.fori_loop(0, NCHUNK, cbody, 0)

        pltpu.sync_copy(dsts_hbm.at[4, wid], ibuf)
        pltpu.sync_copy(srcs_hbm.at[wid], sbuf)

        def abody(i, _):
            r = i // 8
            kk = i % 8
            sl = pl.ds(kk * 16, 16)
            d = ibuf[r, sl]
            s = sbuf[r, sl]
            a = plsc.load_gather(u_tab, [s]) + plsc.load_gather(v_tab, [d])
            a = jnp.maximum(a, 0.01 * a)
            e = jnp.exp(a)
            ebuf[r, sl] = e
            plsc.addupdate_scatter(h4, [d], e)
            return 0
        lax.fori_loop(0, NCHUNK, abody, 0)

        pltpu.sync_copy(ebuf, ev_hbm.at[wid])
        for t in range(5):
            pltpu.sync_copy(hists[t],
                            cnt_hbm.at[pl.ds((wid * 5 + t) * HBINS, HBINS)])

    return k(dsts, srcs, u, vpad)


def _sc_pass2(ptab, pmtab, midx, aidx, dsts2, ev):
    mesh = plsc.VectorSubcoreMesh(core_axis_name="c", subcore_axis_name="s")

    @functools.partial(
        pl.kernel, mesh=mesh,
        compiler_params=pltpu.CompilerParams(use_tc_tiling_on_sc=False,
                                             needs_layout_passes=False),
        out_type=jax.ShapeDtypeStruct((5, 2, ACCR, AD), jnp.float32),
        scratch_types=[
            pltpu.VMEM((WIN, CW), jnp.int32),
            pltpu.VMEM((WIN, CW), jnp.int32),
            pltpu.VMEM((WIN * CW,), jnp.float32),
            pltpu.VMEM((WIN, CW), jnp.int32),
            pltpu.VMEM((WIN, CW), jnp.int32),
            pltpu.VMEM((WIN * CW,), jnp.float32),
            pltpu.VMEM((CW, AD), jnp.float32),
            pltpu.VMEM((CW, AD), jnp.float32),
            pltpu.VMEM_SHARED((ACCR, AD), jnp.float32),
            pltpu.SemaphoreType.DMA,
            pltpu.SemaphoreType.DMA,
            pltpu.SemaphoreType.DMA,
            pltpu.SemaphoreType.DMA,
            pltpu.SemaphoreType.DMA,
        ],
    )
    def k(ptab_hbm, pmtab_hbm, midx_hbm, aidx_hbm, dsts2_hbm, ev_hbm,
          zeros_hbm, s_hbm, src0, dst0, ev0, src1, dst1, ev1, rows0, rows1,
          acc, sem_a, sem_b, sem_c, sem_d, sem_e):
        c = lax.axis_index("c")
        tid = lax.axis_index("s")
        bufs = ((src0, dst0, ev0), (src1, dst1, ev1))
        nwin = NJ // WIN

        def make_scale(ebase0, rows, ev_v):
            def scale(r, _2):
                es = plsc.load_gather(
                    ev_v, [jnp.full((16,), 0, jnp.int32) + (ebase0 + r)])
                for kk in range(8):
                    sl = pl.ds(kk * 16, 16)
                    rows[r, sl] = rows[r, sl] * es
                return 0
            return scale

        def stage_descs(t, w, b):
            sv, dv, evv = bufs[b]
            wbase = tid * NJ + w * WIN
            descs = []
            if t < 4:
                descs.append((midx_hbm.at[c, t, pl.ds(wbase, WIN)], sv))
            else:
                descs.append((aidx_hbm.at[c, pl.ds(wbase, WIN)], sv))
                descs.append((ev_hbm.at[pl.ds(wbase * CW, WIN * CW)], evv))
            descs.append((dsts2_hbm.at[t, pl.ds(wbase, WIN)], dv))
            return descs

        for t in range(5):
            pltpu.sync_copy(zeros_hbm.at[pl.ds(tid * STRIPE, STRIPE)],
                            acc.at[pl.ds(tid * STRIPE, STRIPE)])
            plsc.subcore_barrier()

            tab = ptab_hbm if t < 4 else pmtab_hbm
            for s, d in stage_descs(t, 0, 0):
                pltpu.sync_copy(s, d)

            for w in range(nwin):
                src_v, dst_v, ev_v = bufs[w % 2]
                if w + 1 < nwin:
                    nxt = stage_descs(t, w + 1, (w + 1) % 2)
                    for s, d in nxt:
                        pltpu.async_copy(s, d, sem_e)

                pltpu.async_copy(tab.at[src_v.at[0]], rows0, sem_a)

                def pair(i, _, _t=t, _tab=tab, _sv=src_v, _dv=dst_v,
                         _ev=ev_v):
                    j0 = 2 * i
                    j1 = j0 + 1

                    @pl.when(i > 0)
                    def _():
                        pltpu.make_async_copy(
                            rows1, acc.at[_dv.at[j0 - 1]], sem_d).wait()
                    pltpu.async_copy(_tab.at[_sv.at[j1]], rows1, sem_b)
                    pltpu.make_async_copy(
                        _tab.at[_sv.at[j0]], rows0, sem_a).wait()
                    if _t == 4:
                        lax.fori_loop(0, CW,
                                      make_scale(j0 * CW, rows0, _ev), 0)
                    pltpu.async_copy(rows0, acc.at[_dv.at[j0]], sem_c,
                                     add=True)
                    pltpu.make_async_copy(
                        _tab.at[_sv.at[j1]], rows1, sem_b).wait()
                    if _t == 4:
                        lax.fori_loop(0, CW,
                                      make_scale(j1 * CW, rows1, _ev), 0)
                    pltpu.make_async_copy(
                        rows0, acc.at[_dv.at[j0]], sem_c).wait()

                    @pl.when(i < WIN // 2 - 1)
                    def _():
                        pltpu.async_copy(_tab.at[_sv.at[j0 + 2]], rows0,
                                         sem_a)
                    pltpu.async_copy(rows1, acc.at[_dv.at[j1]], sem_d,
                                     add=True)
                    return 0
                lax.fori_loop(0, WIN // 2, pair, 0)
                pltpu.make_async_copy(
                    rows1, acc.at[dst_v.at[WIN - 1]], sem_d).wait()
                if w + 1 < nwin:
                    for s, d in nxt:
                        pltpu.make_async_copy(s, d, sem_e).wait()

            plsc.subcore_barrier()
            pltpu.sync_copy(acc.at[pl.ds(tid * STRIPE, STRIPE)],
                            s_hbm.at[t, c, pl.ds(tid * STRIPE, STRIPE)])
            plsc.subcore_barrier()

    return k(ptab, pmtab, midx, aidx, dsts2, ev,
             jnp.zeros((ACCR, AD), jnp.float32))


def _post_tc(r_ref, s_ref, c_ref, o_ref):
    cnts = jnp.sum(c_ref[...], axis=2)
    acc = r_ref[0]
    for t in range(4):
        ct = cnts[:, t]
        st = jnp.concatenate([s_ref[t, 0], s_ref[t, 1]], axis=1)
        gate = jnp.where(ct > 0, 1.0, 0.0)[:, None]
        acc = acc + st / jnp.maximum(ct, 1.0)[:, None] + gate * r_ref[t + 1]
    den = cnts[:, 4]
    sa = jnp.concatenate([s_ref[4, 0], s_ref[4, 1]], axis=1)
    gate = jnp.where(den > 0, 1.0, 0.0)[:, None]
    acc = acc + sa / jnp.maximum(den, 1e-30)[:, None] + gate * r_ref[5]
    o_ref[...] = jnp.maximum(acc, 0.0)


def _run_post_tc(r, s, cnt):
    nb = NCELL // RB
    return pl.pallas_call(
        _post_tc,
        grid=(nb,),
        in_specs=[
            pl.BlockSpec((6, RB, H), lambda i: (0, i, 0)),
            pl.BlockSpec((5, 2, RB, AD), lambda i: (0, 0, i, 0)),
            pl.BlockSpec((RB, 5, NW), lambda i: (i, 0, 0)),
        ],
        out_specs=pl.BlockSpec((RB, H), lambda i: (i, 0)),
        out_shape=jax.ShapeDtypeStruct((NCELL, H), jnp.float32),
    )(r, s, cnt)


def kernel(cell_h, cluster_h, edge_diff, edge_lt, edge_gt, edge_contains,
           mc_src, mc_dst, W_diff, W_lt, W_gt, W_contains, W_may_contain,
           W_cell_fc, W_cluster_fc, W_attn, W_upd):
    pad = EP - E
    dsts = jnp.stack([edge_diff[1], edge_lt[1], edge_gt[1],
                      edge_contains[1], mc_dst]).astype(jnp.int32)
    dsts = jnp.pad(dsts, ((0, 0), (0, pad)), constant_values=NCELL)
    dsts = dsts.reshape(5, NW, PW // 128, 128)
    srcs = jnp.pad(mc_src.astype(jnp.int32),
                   (0, pad)).reshape(NW, PW // 128, 128)

    pm, u, cp, cr, fcv = _run_cluster_tc(
        cluster_h, W_diff, W_lt, W_gt, W_contains, W_may_contain, W_upd,
        W_cell_fc, W_cluster_fc, W_attn)
    v = _run_vcell_tc(cell_h, fcv)

    pmtab = pm.reshape(2 * NCLUSTER, AD)
    u1 = u.reshape(NCLUSTER)
    vpad = jnp.pad(v.reshape(NCELL), (0, HBINS - NCELL))

    cnt, evals = _sc_pass1(dsts, srcs, u1, vpad)
    ptab4, r = _run_cell_tc(cell_h, cp, cr, W_upd)
    ptab = ptab4.reshape(4 * 2 * NCELL, AD)

    srcs4 = jnp.stack([edge_diff[0], edge_lt[0], edge_gt[0],
                       edge_contains[0]]).astype(jnp.int32)
    off = ((jnp.arange(4)[None, :] * 2 + jnp.arange(2)[:, None])
           * NCELL).astype(jnp.int32)[:, :, None]
    midx = (srcs4[None] + off).reshape(2, 4, E // CW, CW)
    aidx = (mc_src.astype(jnp.int32)[None]
            + jnp.arange(2, dtype=jnp.int32)[:, None] * NCLUSTER)
    aidx = aidx.reshape(2, E // CW, CW)
    dsts2 = jnp.stack([edge_diff[1], edge_lt[1], edge_gt[1],
                       edge_contains[1], mc_dst]).astype(jnp.int32)
    dsts2 = dsts2.reshape(5, E // CW, CW)
    ev = evals.reshape(EP)[:E]

    s = _sc_pass2(ptab, pmtab, midx, aidx, dsts2, ev)

    cntc = cnt.reshape(NW, 5, HBINS)[:, :, :NCELL].transpose(2, 1, 0)
    return _run_post_tc(r, s, cntc)


# speedup vs baseline: 1.0072x; 1.0072x over previous
import functools

import jax
import jax.numpy as jnp
from jax import lax
from jax.experimental import pallas as pl
from jax.experimental.pallas import tpu as pltpu
from jax.experimental.pallas import tpu_sc as plsc

H = 256
AD = 128
NCELL = 10000
NCLUSTER = 2000
E = 160000
NW = 32
PW = 5120
EP = NW * PW
NCHUNK = PW // 16
HBR = 79
HBINS = HBR * 128
CW = 125
NJ = 80
WIN = 8
ACCR = 10112
STRIPE = ACCR // 16
RB = 1000


def _dg(a, b, ca, cb):
    return lax.dot_general(a, b, dimension_numbers=(((ca,), (cb,)), ((), ())),
                           preferred_element_type=jnp.float32)


def _cluster_tc(ch_ref, wd_ref, wl_ref, wg_ref, wc_ref, wm_ref, wupd_ref,
                wcfc_ref, wclfc_ref, wattn_ref,
                pm_ref, u_ref, cp_ref, cr_ref, fcv_ref):
    ch = ch_ref[...]
    for t, w_ref in enumerate((wd_ref, wl_ref, wg_ref, wc_ref)):
        vs = wupd_ref[:, (t + 1) * H:(t + 2) * H]
        cp_ref[t] = _dg(w_ref[:, :H], vs, 0, 1)
        cr_ref[t] = _dg(w_ref[:, H:], vs, 0, 1)
    v5 = wupd_ref[:, 5 * H:6 * H]
    cm = _dg(wm_ref[:, :H], v5, 0, 1)
    cr_ref[4] = _dg(wm_ref[:, H:], v5, 0, 1)
    pm = _dg(ch, cm, 1, 0)
    pm_ref[0] = pm[:, :AD]
    pm_ref[1] = pm[:, AD:]
    z = _dg(ch, wclfc_ref[...], 1, 1)
    u_ref[...] = _dg(z, wattn_ref[:, :AD], 1, 1)
    fcv_ref[...] = _dg(wcfc_ref[...], wattn_ref[:, AD:], 0, 1)


def _run_cluster_tc(cluster_h, wd, wl, wg, wc, w_may, w_upd,
                    w_cell_fc, w_cluster_fc, w_attn):
    return pl.pallas_call(
        _cluster_tc,
        out_shape=[
            jax.ShapeDtypeStruct((2, NCLUSTER, AD), jnp.float32),
            jax.ShapeDtypeStruct((NCLUSTER, 1), jnp.float32),
            jax.ShapeDtypeStruct((4, H, H), jnp.float32),
            jax.ShapeDtypeStruct((5, H, H), jnp.float32),
            jax.ShapeDtypeStruct((H, 1), jnp.float32),
        ],
    )(cluster_h, wd, wl, wg, wc, w_may, w_upd,
      w_cell_fc, w_cluster_fc, w_attn)


def _vcell_tc(x_ref, fcv_ref, v_ref):
    v_ref[...] = _dg(x_ref[...], fcv_ref[...], 1, 0)


def _run_vcell_tc(cell_h, fcv):
    return pl.pallas_call(
        _vcell_tc,
        out_shape=jax.ShapeDtypeStruct((NCELL, 1), jnp.float32),
    )(cell_h, fcv)


def _cell_tc(x_ref, cp_ref, cr_ref, wupd_ref, p_ref, r_ref):
    x = x_ref[...]
    r_ref[0] = _dg(x, wupd_ref[:, 0:H], 1, 1)
    for t in range(4):
        p = _dg(x, cp_ref[t], 1, 0)
        p_ref[t, 0] = p[:, :AD]
        p_ref[t, 1] = p[:, AD:]
    for t in range(5):
        r_ref[t + 1] = _dg(x, cr_ref[t], 1, 0)


def _run_cell_tc(cell_h, cp, cr, w_upd):
    nb = NCELL // RB
    full = lambda s: pl.BlockSpec(s, lambda i: tuple(0 for _ in s))
    return pl.pallas_call(
        _cell_tc,
        grid=(nb,),
        in_specs=[
            pl.BlockSpec((RB, H), lambda i: (i, 0)),
            full((4, H, H)), full((5, H, H)), full((H, 6 * H)),
        ],
        out_specs=[
            pl.BlockSpec((4, 2, RB, AD), lambda i: (0, 0, i, 0)),
            pl.BlockSpec((6, RB, H), lambda i: (0, i, 0)),
        ],
        out_shape=[
            jax.ShapeDtypeStruct((4, 2, NCELL, AD), jnp.float32),
            jax.ShapeDtypeStruct((6, NCELL, H), jnp.float32),
        ],
    )(cell_h, cp, cr, w_upd)


def _sc_pass1(dsts, srcs, u, vpad):
    mesh = plsc.VectorSubcoreMesh(core_axis_name="c", subcore_axis_name="s")

    @functools.partial(
        pl.kernel, mesh=mesh,
        compiler_params=pltpu.CompilerParams(use_tc_tiling_on_sc=False,
                                             needs_layout_passes=False),
        out_type=[
            jax.ShapeDtypeStruct((NW * 5 * HBINS,), jnp.float32),
            jax.ShapeDtypeStruct((NW, PW // 128, 128), jnp.float32),
        ],
        scratch_types=[
            pltpu.VMEM((PW // 128, 128), jnp.int32),
            pltpu.VMEM((PW // 128, 128), jnp.int32),
            pltpu.VMEM((PW // 128, 128), jnp.float32),
            pltpu.VMEM((NCLUSTER,), jnp.float32),
            pltpu.VMEM((HBINS,), jnp.float32),
            pltpu.VMEM((HBINS,), jnp.float32),
            pltpu.VMEM((HBINS,), jnp.float32),
            pltpu.VMEM((HBINS,), jnp.float32),
            pltpu.VMEM((HBINS,), jnp.float32),
            pltpu.VMEM((HBINS,), jnp.float32),
        ],
    )
    def k(dsts_hbm, srcs_hbm, u_hbm, v_hbm, cnt_hbm, ev_hbm,
          ibuf, sbuf, ebuf, u_tab, v_tab, h0, h1, h2, h3, h4):
        hists = (h0, h1, h2, h3, h4)
        wid = lax.axis_index("s") * 2 + lax.axis_index("c")

        zeros16 = jnp.zeros((16,), jnp.float32)

        def zero_body(i, _):
            for hr in hists:
                hr[pl.ds(i * 16, 16)] = zeros16
            return 0
        lax.fori_loop(0, HBINS // 16, zero_body, 0)

        pltpu.sync_copy(u_hbm, u_tab)
        pltpu.sync_copy(v_hbm, v_tab)

        ones16 = jnp.ones((16,), jnp.float32)
        for t in range(4):
            pltpu.sync_copy(dsts_hbm.at[t, wid], ibuf)

            def cbody(i, _, _t=t):
                r = i // 8
                kk = i % 8
                d = ibuf[r, pl.ds(kk * 16, 16)]
                plsc.addupdate_scatter(hists[_t], [d], ones16)
                return 0
            lax.fori_loop(0, NCHUNK, cbody, 0)

        pltpu.sync_copy(dsts_hbm.at[4, wid], ibuf)
        pltpu.sync_copy(srcs_hbm.at[wid], sbuf)

        def abody(i, _):
            r = i // 8
            kk = i % 8
            sl = pl.ds(kk * 16, 16)
            d = ibuf[r, sl]
            s = sbuf[r, sl]
            a = plsc.load_gather(u_tab, [s]) + plsc.load_gather(v_tab, [d])
            a = jnp.maximum(a, 0.01 * a)
            e = jnp.exp(a)
            ebuf[r, sl] = e
            plsc.addupdate_scatter(h4, [d], e)
            return 0
        lax.fori_loop(0, NCHUNK, abody, 0)

        pltpu.sync_copy(ebuf, ev_hbm.at[wid])
        for t in range(5):
            pltpu.sync_copy(hists[t],
                            cnt_hbm.at[pl.ds((wid * 5 + t) * HBINS, HBINS)])

    return k(dsts, srcs, u, vpad)


def _sc_pass2(ptab, pmtab, midx, aidx, dsts2, ev):
    mesh = plsc.VectorSubcoreMesh(core_axis_name="c", subcore_axis_name="s")

    @functools.partial(
        pl.kernel, mesh=mesh,
        compiler_params=pltpu.CompilerParams(use_tc_tiling_on_sc=False,
                                             needs_layout_passes=False),
        out_type=jax.ShapeDtypeStruct((5, 2, ACCR, AD), jnp.float32),
        scratch_types=[
            pltpu.VMEM((WIN, CW), jnp.int32),
            pltpu.VMEM((WIN, CW), jnp.int32),
            pltpu.VMEM((WIN * CW,), jnp.float32),
            pltpu.VMEM((WIN, CW), jnp.int32),
            pltpu.VMEM((WIN, CW), jnp.int32),
            pltpu.VMEM((WIN * CW,), jnp.float32),
            pltpu.VMEM((CW, AD), jnp.float32),
            pltpu.VMEM((CW, AD), jnp.float32),
            pltpu.VMEM_SHARED((ACCR, AD), jnp.float32),
            pltpu.SemaphoreType.DMA,
            pltpu.SemaphoreType.DMA,
            pltpu.SemaphoreType.DMA,
            pltpu.SemaphoreType.DMA,
            pltpu.SemaphoreType.DMA,
        ],
    )
    def k(ptab_hbm, pmtab_hbm, midx_hbm, aidx_hbm, dsts2_hbm, ev_hbm,
          zeros_hbm, s_hbm, src0, dst0, ev0, src1, dst1, ev1, rows0, rows1,
          acc, sem_a, sem_b, sem_c, sem_d, sem_e):
        c = lax.axis_index("c")
        tid = lax.axis_index("s")
        bufs = ((src0, dst0, ev0), (src1, dst1, ev1))
        nwin = NJ // WIN

        def make_scale(ebase0, rows, ev_v):
            def scale(r, _2):
                es = plsc.load_gather(
                    ev_v, [jnp.full((16,), 0, jnp.int32) + (ebase0 + r)])
                for kk in range(8):
                    sl = pl.ds(kk * 16, 16)
                    rows[r, sl] = rows[r, sl] * es
                return 0
            return scale

        def stage_descs(t, w, b):
            sv, dv, evv = bufs[b]
            wbase = tid * NJ + w * WIN
            descs = []
            if t < 4:
                descs.append((midx_hbm.at[c, t, pl.ds(wbase, WIN)], sv))
            else:
                descs.append((aidx_hbm.at[c, pl.ds(wbase, WIN)], sv))
                descs.append((ev_hbm.at[pl.ds(wbase * CW, WIN * CW)], evv))
            descs.append((dsts2_hbm.at[t, pl.ds(wbase, WIN)], dv))
            return descs

        for t in range(5):
            pltpu.sync_copy(zeros_hbm.at[pl.ds(tid * STRIPE, STRIPE)],
                            acc.at[pl.ds(tid * STRIPE, STRIPE)])
            plsc.subcore_barrier()

            tab = ptab_hbm if t < 4 else pmtab_hbm
            for s, d in stage_descs(t, 0, 0):
                pltpu.sync_copy(s, d)

            for w in range(nwin):
                src_v, dst_v, ev_v = bufs[w % 2]
                if w + 1 < nwin:
                    nxt = stage_descs(t, w + 1, (w + 1) % 2)
                    for s, d in nxt:
                        pltpu.async_copy(s, d, sem_e)

                pltpu.async_copy(tab.at[src_v.at[0]], rows0, sem_a)

                def pair(i, _, _t=t, _tab=tab, _sv=src_v, _dv=dst_v,
                         _ev=ev_v):
                    j0 = 2 * i
                    j1 = j0 + 1

                    @pl.when(i > 0)
                    def _():
                        pltpu.make_async_copy(
                            rows1, acc.at[_dv.at[j0 - 1]], sem_d).wait()
                    pltpu.async_copy(_tab.at[_sv.at[j1]], rows1, sem_b)
                    pltpu.make_async_copy(
                        _tab.at[_sv.at[j0]], rows0, sem_a).wait()
                    if _t == 4:
                        lax.fori_loop(0, CW,
                                      make_scale(j0 * CW, rows0, _ev), 0,
                                      unroll=5)
                    pltpu.async_copy(rows0, acc.at[_dv.at[j0]], sem_c,
                                     add=True)
                    pltpu.make_async_copy(
                        _tab.at[_sv.at[j1]], rows1, sem_b).wait()
                    if _t == 4:
                        lax.fori_loop(0, CW,
                                      make_scale(j1 * CW, rows1, _ev), 0,
                                      unroll=5)
                    pltpu.make_async_copy(
                        rows0, acc.at[_dv.at[j0]], sem_c).wait()

                    @pl.when(i < WIN // 2 - 1)
                    def _():
                        pltpu.async_copy(_tab.at[_sv.at[j0 + 2]], rows0,
                                         sem_a)
                    pltpu.async_copy(rows1, acc.at[_dv.at[j1]], sem_d,
                                     add=True)
                    return 0
                lax.fori_loop(0, WIN // 2, pair, 0)
                pltpu.make_async_copy(
                    rows1, acc.at[dst_v.at[WIN - 1]], sem_d).wait()
                if w + 1 < nwin:
                    for s, d in nxt:
                        pltpu.make_async_copy(s, d, sem_e).wait()

            plsc.subcore_barrier()
            pltpu.sync_copy(acc.at[pl.ds(tid * STRIPE, STRIPE)],
                            s_hbm.at[t, c, pl.ds(tid * STRIPE, STRIPE)])
            plsc.subcore_barrier()

    return k(ptab, pmtab, midx, aidx, dsts2, ev,
             jnp.zeros((ACCR, AD), jnp.float32))


def _post_tc(r_ref, s_ref, c_ref, o_ref):
    cnts = jnp.sum(c_ref[...], axis=2)
    acc = r_ref[0]
    for t in range(4):
        ct = cnts[:, t]
        st = jnp.concatenate([s_ref[t, 0], s_ref[t, 1]], axis=1)
        gate = jnp.where(ct > 0, 1.0, 0.0)[:, None]
        acc = acc + st / jnp.maximum(ct, 1.0)[:, None] + gate * r_ref[t + 1]
    den = cnts[:, 4]
    sa = jnp.concatenate([s_ref[4, 0], s_ref[4, 1]], axis=1)
    gate = jnp.where(den > 0, 1.0, 0.0)[:, None]
    acc = acc + sa / jnp.maximum(den, 1e-30)[:, None] + gate * r_ref[5]
    o_ref[...] = jnp.maximum(acc, 0.0)


def _run_post_tc(r, s, cnt):
    nb = NCELL // RB
    return pl.pallas_call(
        _post_tc,
        grid=(nb,),
        in_specs=[
            pl.BlockSpec((6, RB, H), lambda i: (0, i, 0)),
            pl.BlockSpec((5, 2, RB, AD), lambda i: (0, 0, i, 0)),
            pl.BlockSpec((RB, 5, NW), lambda i: (i, 0, 0)),
        ],
        out_specs=pl.BlockSpec((RB, H), lambda i: (i, 0)),
        out_shape=jax.ShapeDtypeStruct((NCELL, H), jnp.float32),
    )(r, s, cnt)


def kernel(cell_h, cluster_h, edge_diff, edge_lt, edge_gt, edge_contains,
           mc_src, mc_dst, W_diff, W_lt, W_gt, W_contains, W_may_contain,
           W_cell_fc, W_cluster_fc, W_attn, W_upd):
    pad = EP - E
    dsts = jnp.stack([edge_diff[1], edge_lt[1], edge_gt[1],
                      edge_contains[1], mc_dst]).astype(jnp.int32)
    dsts = jnp.pad(dsts, ((0, 0), (0, pad)), constant_values=NCELL)
    dsts = dsts.reshape(5, NW, PW // 128, 128)
    srcs = jnp.pad(mc_src.astype(jnp.int32),
                   (0, pad)).reshape(NW, PW // 128, 128)

    pm, u, cp, cr, fcv = _run_cluster_tc(
        cluster_h, W_diff, W_lt, W_gt, W_contains, W_may_contain, W_upd,
        W_cell_fc, W_cluster_fc, W_attn)
    v = _run_vcell_tc(cell_h, fcv)

    pmtab = pm.reshape(2 * NCLUSTER, AD)
    u1 = u.reshape(NCLUSTER)
    vpad = jnp.pad(v.reshape(NCELL), (0, HBINS - NCELL))

    cnt, evals = _sc_pass1(dsts, srcs, u1, vpad)
    ptab4, r = _run_cell_tc(cell_h, cp, cr, W_upd)
    ptab = ptab4.reshape(4 * 2 * NCELL, AD)

    srcs4 = jnp.stack([edge_diff[0], edge_lt[0], edge_gt[0],
                       edge_contains[0]]).astype(jnp.int32)
    off = ((jnp.arange(4)[None, :] * 2 + jnp.arange(2)[:, None])
           * NCELL).astype(jnp.int32)[:, :, None]
    midx = (srcs4[None] + off).reshape(2, 4, E // CW, CW)
    aidx = (mc_src.astype(jnp.int32)[None]
            + jnp.arange(2, dtype=jnp.int32)[:, None] * NCLUSTER)
    aidx = aidx.reshape(2, E // CW, CW)
    dsts2 = jnp.stack([edge_diff[1], edge_lt[1], edge_gt[1],
                       edge_contains[1], mc_dst]).astype(jnp.int32)
    dsts2 = dsts2.reshape(5, E // CW, CW)
    ev = evals.reshape(EP)[:E]

    s = _sc_pass2(ptab, pmtab, midx, aidx, dsts2, ev)

    cntc = cnt.reshape(NW, 5, HBINS)[:, :, :NCELL].transpose(2, 1, 0)
    return _run_post_tc(r, s, cntc)
